# baseline (speedup 1.0000x reference)
import functools

import jax
import jax.numpy as jnp
from jax import lax
from jax.experimental import pallas as pl
from jax.experimental.pallas import tpu as pltpu
from jax.experimental.pallas import tpu_sc as plsc

N_NODES = 10000
N_EDGES = 320000
NC, NS = 2, 16
NW = NC * NS
CHUNK = 128
N_CHUNKS = 80
E_PAD = NW * N_CHUNKS * CHUNK
N_ACC = 10240
ZROWS = N_ACC // NS


def _make_agg(F, tc_tiling=True, NBUF=4, NSLOT=8):
    mesh = plsc.VectorSubcoreMesh(core_axis_name="c", subcore_axis_name="s")

    @functools.partial(
        pl.kernel,
        out_type=jax.ShapeDtypeStruct((NC * N_ACC, F), jnp.float32),
        mesh=mesh,
        compiler_params=pltpu.CompilerParams(use_tc_tiling_on_sc=tc_tiling),
        scratch_types=(
            [pltpu.VMEM_SHARED((N_ACC, F), jnp.float32),
             pltpu.VMEM((NSLOT, 2, CHUNK), jnp.int32)]
            + [pltpu.VMEM((CHUNK, F), jnp.float32) for _ in range(NBUF)]
            + [pltpu.SemaphoreType.DMA for _ in range(NSLOT + NBUF)]
        ),
    )
    def agg(z_hbm, idx_hbm, zeros_hbm, out_hbm, acc, ring, *rest):
        bufs = rest[:NBUF]
        isems = rest[NBUF:NBUF + NSLOT]
        gsems = rest[NBUF + NSLOT:]
        c = lax.axis_index("c")
        s = lax.axis_index("s")
        wid = c * NS + s
        pltpu.sync_copy(zeros_hbm, acc.at[pl.ds(s * ZROWS, ZROWS)])
        plsc.subcore_barrier()

        def idx_fetch(chunk, slot):
            pltpu.async_copy(idx_hbm.at[wid, chunk], ring.at[slot],
                             isems[slot])

        def idx_wait(chunk, slot):
            pltpu.make_async_copy(idx_hbm.at[wid, chunk], ring.at[slot],
                                  isems[slot]).wait()

        def gather(slot, b):
            pltpu.async_copy(z_hbm.at[ring.at[slot, 0]], bufs[b], gsems[b])

        def gather_wait(slot, b):
            pltpu.make_async_copy(z_hbm.at[ring.at[slot, 0]], bufs[b],
                                  gsems[b]).wait()

        for k in range(NSLOT):
            idx_fetch(k, k)
        for k in range(NBUF):
            idx_wait(k, k)
            gather(k, k)

        def body(jj, carry):
            j = jj * NSLOT
            for k in range(NSLOT):
                b = k % NBUF
                gather_wait(k, b)
                pltpu.sync_copy(bufs[b], acc.at[ring.at[k, 1]], add=True)

                @pl.when(j + k + NSLOT < N_CHUNKS)
                def _():
                    idx_fetch(j + k + NSLOT, k)

                @pl.when(j + k + NBUF < N_CHUNKS)
                def _():
                    s2 = (k + NBUF) % NSLOT
                    idx_wait(j + k + NBUF, s2)
                    gather(s2, b)

            return carry

        lax.fori_loop(0, N_CHUNKS // NSLOT, body, 0)
        plsc.subcore_barrier()
        pltpu.sync_copy(
            acc.at[pl.ds(s * ZROWS, ZROWS)],
            out_hbm.at[pl.ds(c * N_ACC + s * ZROWS, ZROWS)],
        )

    return agg


_agg128 = _make_agg(128, NBUF=2, NSLOT=8)
_agg64 = _make_agg(64, tc_tiling=False, NBUF=8, NSLOT=16)


def _make_first(R, Fin, Fout):
    def body(x_ref, w_ref, o_ref):
        o_ref[...] = jnp.dot(x_ref[...], w_ref[...],
                             preferred_element_type=jnp.float32)

    return pl.pallas_call(
        body,
        grid=(N_NODES // R,),
        in_specs=[
            pl.BlockSpec((R, Fin), lambda i: (i, 0)),
            pl.BlockSpec((Fin, Fout), lambda i: (0, 0)),
        ],
        out_specs=pl.BlockSpec((R, Fout), lambda i: (i, 0)),
        out_shape=jax.ShapeDtypeStruct((N_NODES, Fout), jnp.float32),
    )


def _make_mid(R, Fa, Fb, Fc):
    def body(z_ref, p_ref, b1_ref, wb_ref, bb_ref, wn_ref, o_ref):
        t = jnp.maximum(z_ref[...] + p_ref[0] + p_ref[1] + b1_ref[...], 0.0)
        h = jnp.dot(t, wb_ref[...], preferred_element_type=jnp.float32)
        h = jnp.maximum(h + bb_ref[...], 0.0)
        o_ref[...] = jnp.dot(h, wn_ref[...], preferred_element_type=jnp.float32)

    return pl.pallas_call(
        body,
        grid=(N_NODES // R,),
        in_specs=[
            pl.BlockSpec((R, Fa), lambda i: (i, 0)),
            pl.BlockSpec((2, R, Fa), lambda i: (0, i, 0)),
            pl.BlockSpec((1, Fa), lambda i: (0, 0)),
            pl.BlockSpec((Fa, Fb), lambda i: (0, 0)),
            pl.BlockSpec((1, Fb), lambda i: (0, 0)),
            pl.BlockSpec((Fb, Fc), lambda i: (0, 0)),
        ],
        out_specs=pl.BlockSpec((R, Fc), lambda i: (i, 0)),
        out_shape=jax.ShapeDtypeStruct((N_NODES, Fc), jnp.float32),
    )


def _make_last(R, Fa, Fb):
    def body(z_ref, p_ref, b1_ref, w_ref, b_ref, o_ref):
        t = jnp.maximum(z_ref[...] + p_ref[0] + p_ref[1] + b1_ref[...], 0.0)
        h = jnp.dot(t, w_ref[...], preferred_element_type=jnp.float32)
        o_ref[...] = h + b_ref[...]

    return pl.pallas_call(
        body,
        grid=(N_NODES // R,),
        in_specs=[
            pl.BlockSpec((R, Fa), lambda i: (i, 0)),
            pl.BlockSpec((2, R, Fa), lambda i: (0, i, 0)),
            pl.BlockSpec((1, Fa), lambda i: (0, 0)),
            pl.BlockSpec((Fa, Fb), lambda i: (0, 0)),
            pl.BlockSpec((1, Fb), lambda i: (0, 0)),
        ],
        out_specs=pl.BlockSpec((R, Fb), lambda i: (i, 0)),
        out_shape=jax.ShapeDtypeStruct((N_NODES, Fb), jnp.float32),
    )


_R = 2000
_first = _make_first(_R, 128, 64)
_mid1 = _make_mid(_R, 64, 128, 128)
_mid2 = _make_mid(_R, 128, 64, 64)
_last = _make_last(_R, 64, 128)


def kernel(x, edge_index, W11, b11, W12, b12, W21, b21, W22, b22,
           W31, b31, W32, b32):
    src = edge_index[0]
    dst = edge_index[1]
    pad = E_PAD - N_EDGES
    src_p = jnp.concatenate(
        [src, jnp.zeros((pad,), jnp.int32)]).reshape(NW, N_CHUNKS, 1, CHUNK)
    dst_p = jnp.concatenate(
        [dst, jnp.full((pad,), N_NODES, jnp.int32)]).reshape(NW, N_CHUNKS, 1, CHUNK)
    idx_p = jnp.concatenate([src_p, dst_p], axis=2)
    zeros128 = jnp.zeros((ZROWS, 128), jnp.float32)
    zeros64 = jnp.zeros((ZROWS, 64), jnp.float32)
    b11r = b11.reshape(1, -1)
    b12r = b12.reshape(1, -1)
    b21r = b21.reshape(1, -1)
    b22r = b22.reshape(1, -1)
    b31r = b31.reshape(1, -1)
    b32r = b32.reshape(1, -1)

    z1 = _first(x, W11)
    p1 = _agg64(z1, idx_p, zeros64).reshape(2, N_ACC, 64)
    z2 = _mid1(z1, p1, b11r, W12, b12r, W21)
    p2 = _agg128(z2, idx_p, zeros128).reshape(2, N_ACC, 128)
    z3 = _mid2(z2, p2, b21r, W22, b22r, W31)
    p3 = _agg64(z3, idx_p, zeros64).reshape(2, N_ACC, 64)
    out = _last(z3, p3, b31r, W32, b32r)
    return out


# speedup vs baseline: 1.0165x; 1.0165x over previous
import functools

import jax
import jax.numpy as jnp
from jax import lax
from jax.experimental import pallas as pl
from jax.experimental.pallas import tpu as pltpu
from jax.experimental.pallas import tpu_sc as plsc

N_NODES = 10000
N_EDGES = 320000
NC, NS = 2, 16
NW = NC * NS
CHUNK = 128
N_CHUNKS = 80
E_PAD = NW * N_CHUNKS * CHUNK
N_ACC = 10240
ZROWS = N_ACC // NS


def _make_agg(F, tc_tiling=True, NBUF=4, NSLOT=8):
    mesh = plsc.VectorSubcoreMesh(core_axis_name="c", subcore_axis_name="s")

    @functools.partial(
        pl.kernel,
        out_type=jax.ShapeDtypeStruct((NC * N_ACC, F), jnp.float32),
        mesh=mesh,
        compiler_params=pltpu.CompilerParams(use_tc_tiling_on_sc=tc_tiling),
        scratch_types=(
            [pltpu.VMEM_SHARED((N_ACC, F), jnp.float32),
             pltpu.VMEM((NSLOT, 2, CHUNK), jnp.int32)]
            + [pltpu.VMEM((CHUNK, F), jnp.float32) for _ in range(NBUF)]
            + [pltpu.SemaphoreType.DMA for _ in range(NSLOT + NBUF + 1)]
        ),
    )
    def agg(z_hbm, idx_hbm, zeros_hbm, out_hbm, acc, ring, *rest):
        bufs = rest[:NBUF]
        isems = rest[NBUF:NBUF + NSLOT]
        gsems = rest[NBUF + NSLOT:NBUF + NSLOT + NBUF]
        zsem = rest[NBUF + NSLOT + NBUF]
        c = lax.axis_index("c")
        s = lax.axis_index("s")
        wid = c * NS + s
        pltpu.async_copy(zeros_hbm, acc.at[pl.ds(s * ZROWS, ZROWS)], zsem)

        def idx_fetch(chunk, slot):
            pltpu.async_copy(idx_hbm.at[wid, chunk], ring.at[slot],
                             isems[slot])

        def idx_wait(chunk, slot):
            pltpu.make_async_copy(idx_hbm.at[wid, chunk], ring.at[slot],
                                  isems[slot]).wait()

        def gather(slot, b):
            pltpu.async_copy(z_hbm.at[ring.at[slot, 0]], bufs[b], gsems[b])

        def gather_wait(slot, b):
            pltpu.make_async_copy(z_hbm.at[ring.at[slot, 0]], bufs[b],
                                  gsems[b]).wait()

        for k in range(NSLOT):
            idx_fetch(k, k)
        for k in range(NBUF):
            idx_wait(k, k)
            gather(k, k)
        pltpu.make_async_copy(zeros_hbm, acc.at[pl.ds(s * ZROWS, ZROWS)],
                              zsem).wait()
        plsc.subcore_barrier()

        def body(jj, carry):
            j = jj * NSLOT
            for k in range(NSLOT):
                b = k % NBUF
                gather_wait(k, b)
                pltpu.sync_copy(bufs[b], acc.at[ring.at[k, 1]], add=True)

                @pl.when(j + k + NSLOT < N_CHUNKS)
                def _():
                    idx_fetch(j + k + NSLOT, k)

                @pl.when(j + k + NBUF < N_CHUNKS)
                def _():
                    s2 = (k + NBUF) % NSLOT
                    idx_wait(j + k + NBUF, s2)
                    gather(s2, b)

            return carry

        lax.fori_loop(0, N_CHUNKS // NSLOT, body, 0)
        plsc.subcore_barrier()
        pltpu.sync_copy(
            acc.at[pl.ds(s * ZROWS, ZROWS)],
            out_hbm.at[pl.ds(c * N_ACC + s * ZROWS, ZROWS)],
        )

    return agg


_agg128 = _make_agg(128, NBUF=2, NSLOT=8)
_agg64 = _make_agg(64, tc_tiling=False, NBUF=4, NSLOT=8)


def _make_first(R, Fin, Fout):
    def body(x_ref, w_ref, o_ref):
        o_ref[...] = jnp.dot(x_ref[...], w_ref[...],
                             preferred_element_type=jnp.float32)

    return pl.pallas_call(
        body,
        grid=(N_NODES // R,),
        in_specs=[
            pl.BlockSpec((R, Fin), lambda i: (i, 0)),
            pl.BlockSpec((Fin, Fout), lambda i: (0, 0)),
        ],
        out_specs=pl.BlockSpec((R, Fout), lambda i: (i, 0)),
        out_shape=jax.ShapeDtypeStruct((N_NODES, Fout), jnp.float32),
    )


def _make_mid(R, Fa, Fb, Fc):
    def body(z_ref, p_ref, b1_ref, wb_ref, bb_ref, wn_ref, o_ref):
        t = jnp.maximum(z_ref[...] + p_ref[0] + p_ref[1] + b1_ref[...], 0.0)
        h = jnp.dot(t, wb_ref[...], preferred_element_type=jnp.float32)
        h = jnp.maximum(h + bb_ref[...], 0.0)
        o_ref[...] = jnp.dot(h, wn_ref[...], preferred_element_type=jnp.float32)

    return pl.pallas_call(
        body,
        grid=(N_NODES // R,),
        in_specs=[
            pl.BlockSpec((R, Fa), lambda i: (i, 0)),
            pl.BlockSpec((2, R, Fa), lambda i: (0, i, 0)),
            pl.BlockSpec((1, Fa), lambda i: (0, 0)),
            pl.BlockSpec((Fa, Fb), lambda i: (0, 0)),
            pl.BlockSpec((1, Fb), lambda i: (0, 0)),
            pl.BlockSpec((Fb, Fc), lambda i: (0, 0)),
        ],
        out_specs=pl.BlockSpec((R, Fc), lambda i: (i, 0)),
        out_shape=jax.ShapeDtypeStruct((N_NODES, Fc), jnp.float32),
    )


def _make_last(R, Fa, Fb):
    def body(z_ref, p_ref, b1_ref, w_ref, b_ref, o_ref):
        t = jnp.maximum(z_ref[...] + p_ref[0] + p_ref[1] + b1_ref[...], 0.0)
        h = jnp.dot(t, w_ref[...], preferred_element_type=jnp.float32)
        o_ref[...] = h + b_ref[...]

    return pl.pallas_call(
        body,
        grid=(N_NODES // R,),
        in_specs=[
            pl.BlockSpec((R, Fa), lambda i: (i, 0)),
            pl.BlockSpec((2, R, Fa), lambda i: (0, i, 0)),
            pl.BlockSpec((1, Fa), lambda i: (0, 0)),
            pl.BlockSpec((Fa, Fb), lambda i: (0, 0)),
            pl.BlockSpec((1, Fb), lambda i: (0, 0)),
        ],
        out_specs=pl.BlockSpec((R, Fb), lambda i: (i, 0)),
        out_shape=jax.ShapeDtypeStruct((N_NODES, Fb), jnp.float32),
    )


_R = 2000
_first = _make_first(_R, 128, 64)
_mid1 = _make_mid(_R, 64, 128, 128)
_mid2 = _make_mid(_R, 128, 64, 64)
_last = _make_last(_R, 64, 128)


def kernel(x, edge_index, W11, b11, W12, b12, W21, b21, W22, b22,
           W31, b31, W32, b32):
    src = edge_index[0]
    dst = edge_index[1]
    pad = E_PAD - N_EDGES
    src_p = jnp.concatenate(
        [src, jnp.zeros((pad,), jnp.int32)]).reshape(NW, N_CHUNKS, 1, CHUNK)
    dst_p = jnp.concatenate(
        [dst, jnp.full((pad,), N_NODES, jnp.int32)]).reshape(NW, N_CHUNKS, 1, CHUNK)
    idx_p = jnp.concatenate([src_p, dst_p], axis=2)
    zeros128 = jnp.zeros((ZROWS, 128), jnp.float32)
    zeros64 = jnp.zeros((ZROWS, 64), jnp.float32)
    b11r = b11.reshape(1, -1)
    b12r = b12.reshape(1, -1)
    b21r = b21.reshape(1, -1)
    b22r = b22.reshape(1, -1)
    b31r = b31.reshape(1, -1)
    b32r = b32.reshape(1, -1)

    z1 = _first(x, W11)
    p1 = _agg64(z1, idx_p, zeros64).reshape(2, N_ACC, 64)
    z2 = _mid1(z1, p1, b11r, W12, b12r, W21)
    p2 = _agg128(z2, idx_p, zeros128).reshape(2, N_ACC, 128)
    z3 = _mid2(z2, p2, b21r, W22, b22r, W31)
    p3 = _agg64(z3, idx_p, zeros64).reshape(2, N_ACC, 64)
    out = _last(z3, p3, b31r, W32, b32r)
    return out
